# Initial kernel scaffold; baseline (speedup 1.0000x reference)
import jax
import jax.numpy as jnp
from jax.experimental import pallas as pl


def kernel(x_mirna, x_disease, edge_label_index, W_mirna, b_mirna, W_disease, b_disease, W_cls, b_cls):
    raise NotImplementedError("write your pallas kernel here")


# speedup vs baseline: 17.3088x; 17.3088x over previous
import functools

import jax
import jax.numpy as jnp
from jax import lax
from jax.experimental import pallas as pl
from jax.experimental.pallas import tpu as pltpu
from jax.experimental.pallas import tpu_sc as plsc

N_NODES = 10000
E_EDGES = 320000
DIM = 128
LANES = 16
NUM_WORKERS = 32
E_PER_W = E_EDGES // NUM_WORKERS
ROW_BLOCK = 1000


def _scores_body(x_ref, w_ref, wc_ref, b_ref, b0_ref, o_ref):
    v = jnp.dot(w_ref[:], wc_ref[:], preferred_element_type=jnp.float32)
    c = jnp.dot(b_ref[:], wc_ref[:], preferred_element_type=jnp.float32)
    s = jnp.dot(x_ref[:], v, preferred_element_type=jnp.float32)
    o_ref[:] = s + c + b0_ref[:]


def _node_scores(x, w, wc, b, b0):
    n, f = x.shape
    return pl.pallas_call(
        _scores_body,
        grid=(n // ROW_BLOCK,),
        in_specs=[
            pl.BlockSpec((ROW_BLOCK, f), lambda i: (i, 0)),
            pl.BlockSpec((f, DIM), lambda i: (0, 0)),
            pl.BlockSpec((DIM, 1), lambda i: (0, 0)),
            pl.BlockSpec((1, DIM), lambda i: (0, 0)),
            pl.BlockSpec((1, 1), lambda i: (0, 0)),
        ],
        out_specs=pl.BlockSpec((ROW_BLOCK, 1), lambda i: (i, 0)),
        out_shape=jax.ShapeDtypeStruct((n, 1), jnp.float32),
    )(x, w, wc, b, b0)


def _edge_body(sm_hbm, sd_hbm, src_hbm, dst_hbm, out_hbm,
               sm_v, sd_v, src_v, dst_v, out_v):
    wid = lax.axis_index("s") * 2 + lax.axis_index("c")
    base = wid * E_PER_W
    pltpu.sync_copy(sm_hbm, sm_v)
    pltpu.sync_copy(sd_hbm, sd_v)
    pltpu.sync_copy(src_hbm.at[pl.ds(base, E_PER_W)], src_v)
    pltpu.sync_copy(dst_hbm.at[pl.ds(base, E_PER_W)], dst_v)

    def body(i, carry):
        off = i * LANES
        si = src_v[pl.ds(off, LANES)]
        di = dst_v[pl.ds(off, LANES)]
        a = plsc.load_gather(sm_v, [si])
        b = plsc.load_gather(sd_v, [di])
        z = a + b
        out_v[pl.ds(off, LANES)] = 1.0 / (1.0 + jnp.exp(-z))
        return carry

    lax.fori_loop(0, E_PER_W // LANES, body, 0)
    pltpu.sync_copy(out_v, out_hbm.at[pl.ds(base, E_PER_W)])


@functools.cache
def _edge_kernel():
    return pl.kernel(
        _edge_body,
        mesh=plsc.VectorSubcoreMesh(core_axis_name="c", subcore_axis_name="s"),
        compiler_params=pltpu.CompilerParams(needs_layout_passes=False),
        out_type=jax.ShapeDtypeStruct((E_EDGES,), jnp.float32),
        scratch_types=[
            pltpu.VMEM((N_NODES,), jnp.float32),
            pltpu.VMEM((N_NODES,), jnp.float32),
            pltpu.VMEM((E_PER_W,), jnp.int32),
            pltpu.VMEM((E_PER_W,), jnp.int32),
            pltpu.VMEM((E_PER_W,), jnp.float32),
        ],
    )


def kernel(x_mirna, x_disease, edge_label_index, W_mirna, b_mirna,
           W_disease, b_disease, W_cls, b_cls):
    wc_m = W_cls[:DIM]
    wc_d = W_cls[DIM:]
    b0 = b_cls.astype(jnp.float32).reshape(1, 1)
    zero = jnp.zeros((1, 1), jnp.float32)
    s_m = _node_scores(x_mirna, W_mirna, wc_m, b_mirna.reshape(1, DIM), b0)
    s_d = _node_scores(x_disease, W_disease, wc_d, b_disease.reshape(1, DIM), zero)
    src = edge_label_index[0].astype(jnp.int32)
    dst = edge_label_index[1].astype(jnp.int32)
    return _edge_kernel()(s_m.reshape(-1), s_d.reshape(-1), src, dst)


# speedup vs baseline: 24.4563x; 1.4129x over previous
import functools

import jax
import jax.numpy as jnp
from jax import lax
from jax.experimental import pallas as pl
from jax.experimental.pallas import tpu as pltpu
from jax.experimental.pallas import tpu_sc as plsc

N_NODES = 10000
E_EDGES = 320000
DIM = 128
LANES = 16
UNROLL = 5
NUM_WORKERS = 32
E_PER_W = E_EDGES // NUM_WORKERS
ROW_BLOCK = 2000


def _scores_body(xm_ref, xd_ref, wm_ref, wd_ref, wcm_ref, wcd_ref,
                 bm_ref, bd_ref, b0_ref, om_ref, od_ref):
    f32 = jnp.float32
    vm = jnp.dot(wm_ref[:], wcm_ref[:], preferred_element_type=f32)
    vd = jnp.dot(wd_ref[:], wcd_ref[:], preferred_element_type=f32)
    cm = jnp.dot(bm_ref[:], wcm_ref[:], preferred_element_type=f32)
    cd = jnp.dot(bd_ref[:], wcd_ref[:], preferred_element_type=f32)
    dn = (((0,), (1,)), ((), ()))
    sm = lax.dot_general(vm, xm_ref[:], dn, preferred_element_type=f32)
    sd = lax.dot_general(vd, xd_ref[:], dn, preferred_element_type=f32)
    om_ref[:] = (sm + cm + b0_ref[:]).reshape(1, 1, ROW_BLOCK)
    od_ref[:] = (sd + cd).reshape(1, 1, ROW_BLOCK)


def _node_scores(xm, xd, wm, wd, wcm, wcd, bm, bd, b0):
    n = xm.shape[0]
    fm, fd = xm.shape[1], xd.shape[1]
    return pl.pallas_call(
        _scores_body,
        grid=(n // ROW_BLOCK,),
        in_specs=[
            pl.BlockSpec((ROW_BLOCK, fm), lambda i: (i, 0)),
            pl.BlockSpec((ROW_BLOCK, fd), lambda i: (i, 0)),
            pl.BlockSpec((fm, DIM), lambda i: (0, 0)),
            pl.BlockSpec((fd, DIM), lambda i: (0, 0)),
            pl.BlockSpec((DIM, 1), lambda i: (0, 0)),
            pl.BlockSpec((DIM, 1), lambda i: (0, 0)),
            pl.BlockSpec((1, DIM), lambda i: (0, 0)),
            pl.BlockSpec((1, DIM), lambda i: (0, 0)),
            pl.BlockSpec((1, 1), lambda i: (0, 0)),
        ],
        out_specs=[
            pl.BlockSpec((1, 1, ROW_BLOCK), lambda i: (i, 0, 0)),
            pl.BlockSpec((1, 1, ROW_BLOCK), lambda i: (i, 0, 0)),
        ],
        out_shape=[
            jax.ShapeDtypeStruct((n // ROW_BLOCK, 1, ROW_BLOCK), jnp.float32),
            jax.ShapeDtypeStruct((n // ROW_BLOCK, 1, ROW_BLOCK), jnp.float32),
        ],
    )(xm, xd, wm, wd, wcm, wcd, bm, bd, b0)


def _edge_body(sm_hbm, sd_hbm, ei_hbm, out_hbm,
               sm_v, sd_v, src_v, dst_v, out_v):
    wid = lax.axis_index("s") * 2 + lax.axis_index("c")
    base = wid * E_PER_W
    pltpu.sync_copy(sm_hbm, sm_v)
    pltpu.sync_copy(sd_hbm, sd_v)
    pltpu.sync_copy(ei_hbm.at[0, pl.ds(base, E_PER_W)], src_v)
    pltpu.sync_copy(ei_hbm.at[1, pl.ds(base, E_PER_W)], dst_v)

    def body(i, carry):
        chunk = i * (LANES * UNROLL)
        for u in range(UNROLL):
            off = chunk + u * LANES
            si = src_v[pl.ds(off, LANES)]
            di = dst_v[pl.ds(off, LANES)]
            a = plsc.load_gather(sm_v, [si])
            b = plsc.load_gather(sd_v, [di])
            z = a + b
            out_v[pl.ds(off, LANES)] = 1.0 / (1.0 + jnp.exp(-z))
        return carry

    lax.fori_loop(0, E_PER_W // (LANES * UNROLL), body, 0)
    pltpu.sync_copy(out_v, out_hbm.at[pl.ds(base, E_PER_W)])


@functools.cache
def _edge_kernel():
    return pl.kernel(
        _edge_body,
        mesh=plsc.VectorSubcoreMesh(core_axis_name="c", subcore_axis_name="s"),
        compiler_params=pltpu.CompilerParams(
            needs_layout_passes=False, use_tc_tiling_on_sc=False),
        out_type=jax.ShapeDtypeStruct((E_EDGES,), jnp.float32),
        scratch_types=[
            pltpu.VMEM((N_NODES,), jnp.float32),
            pltpu.VMEM((N_NODES,), jnp.float32),
            pltpu.VMEM((E_PER_W,), jnp.int32),
            pltpu.VMEM((E_PER_W,), jnp.int32),
            pltpu.VMEM((E_PER_W,), jnp.float32),
        ],
    )


def kernel(x_mirna, x_disease, edge_label_index, W_mirna, b_mirna,
           W_disease, b_disease, W_cls, b_cls):
    wc_m = W_cls[:DIM]
    wc_d = W_cls[DIM:]
    b0 = b_cls.astype(jnp.float32).reshape(1, 1)
    s_m, s_d = _node_scores(x_mirna, x_disease, W_mirna, W_disease,
                            wc_m, wc_d, b_mirna.reshape(1, DIM),
                            b_disease.reshape(1, DIM), b0)
    ei = edge_label_index.astype(jnp.int32)
    return _edge_kernel()(s_m.reshape(-1), s_d.reshape(-1), ei)


# speedup vs baseline: 25.3748x; 1.0376x over previous
import functools

import jax
import jax.numpy as jnp
from jax import lax
from jax.experimental import pallas as pl
from jax.experimental.pallas import tpu as pltpu
from jax.experimental.pallas import tpu_sc as plsc

N_NODES = 10000
E_EDGES = 320000
DIM = 128
LANES = 16
UNROLL = 5
NUM_WORKERS = 32
E_PER_W = E_EDGES // NUM_WORKERS
ROW_BLOCK = 2000


def _scores_body(xm_ref, xd_ref, wm_ref, wd_ref, wcm_ref, wcd_ref,
                 bm_ref, bd_ref, b0_ref, om_ref, od_ref):
    f32 = jnp.float32
    i = pl.program_id(0)
    vm = jnp.dot(wm_ref[:], wcm_ref[:], preferred_element_type=f32)
    vd = jnp.dot(wd_ref[:], wcd_ref[:], preferred_element_type=f32)
    cm = jnp.dot(bm_ref[:], wcm_ref[:], preferred_element_type=f32)
    cd = jnp.dot(bd_ref[:], wcd_ref[:], preferred_element_type=f32)
    dn = (((0,), (1,)), ((), ()))
    sm = lax.dot_general(vm, xm_ref[:], dn, preferred_element_type=f32)
    sd = lax.dot_general(vd, xd_ref[:], dn, preferred_element_type=f32)
    om_ref[:] = (sm + cm + b0_ref[:]).reshape(1, 1, ROW_BLOCK)
    od_ref[:] = (sd + cd).reshape(1, 1, ROW_BLOCK)


def _node_scores(xm, xd, wm, wd, wcm, wcd, bm, bd, b0):
    n = xm.shape[0]
    fm, fd = xm.shape[1], xd.shape[1]
    return pl.pallas_call(
        _scores_body,
        grid=(n // ROW_BLOCK,),
        in_specs=[
            pl.BlockSpec((ROW_BLOCK, fm), lambda i: (i, 0)),
            pl.BlockSpec((ROW_BLOCK, fd), lambda i: (i, 0)),
            pl.BlockSpec((fm, DIM), lambda i: (0, 0)),
            pl.BlockSpec((fd, DIM), lambda i: (0, 0)),
            pl.BlockSpec((DIM, 1), lambda i: (0, 0)),
            pl.BlockSpec((DIM, 1), lambda i: (0, 0)),
            pl.BlockSpec((1, DIM), lambda i: (0, 0)),
            pl.BlockSpec((1, DIM), lambda i: (0, 0)),
            pl.BlockSpec((1, 1), lambda i: (0, 0)),
        ],
        out_specs=[
            pl.BlockSpec((1, 1, ROW_BLOCK), lambda i: (i, 0, 0)),
            pl.BlockSpec((1, 1, ROW_BLOCK), lambda i: (i, 0, 0)),
        ],
        out_shape=[
            jax.ShapeDtypeStruct((n // ROW_BLOCK, 1, ROW_BLOCK), jnp.float32),
            jax.ShapeDtypeStruct((n // ROW_BLOCK, 1, ROW_BLOCK), jnp.float32),
        ],
    )(xm, xd, wm, wd, wcm, wcd, bm, bd, b0)


def _edge_body(sm_hbm, sd_hbm, ei_hbm, out_hbm,
               sm_v, sd_v, src_v, dst_v, out_v, sem):
    wid = lax.axis_index("s") * 2 + lax.axis_index("c")
    base = wid * E_PER_W
    c1 = pltpu.async_copy(sm_hbm, sm_v, sem)
    c2 = pltpu.async_copy(sd_hbm, sd_v, sem)
    c3 = pltpu.async_copy(ei_hbm.at[0, pl.ds(base, E_PER_W)], src_v, sem)
    c4 = pltpu.async_copy(ei_hbm.at[1, pl.ds(base, E_PER_W)], dst_v, sem)
    c1.wait()
    c2.wait()
    c3.wait()
    c4.wait()

    def body(i, carry):
        chunk = i * (LANES * UNROLL)
        for u in range(UNROLL):
            off = chunk + u * LANES
            si = src_v[pl.ds(off, LANES)]
            di = dst_v[pl.ds(off, LANES)]
            a = plsc.load_gather(sm_v, [si])
            b = plsc.load_gather(sd_v, [di])
            z = a + b
            out_v[pl.ds(off, LANES)] = 1.0 / (1.0 + jnp.exp(-z))
        return carry

    lax.fori_loop(0, E_PER_W // (LANES * UNROLL), body, 0)
    pltpu.sync_copy(out_v, out_hbm.at[pl.ds(base, E_PER_W)])


@functools.cache
def _edge_kernel():
    return pl.kernel(
        _edge_body,
        mesh=plsc.VectorSubcoreMesh(core_axis_name="c", subcore_axis_name="s"),
        compiler_params=pltpu.CompilerParams(
            needs_layout_passes=False, use_tc_tiling_on_sc=False),
        out_type=jax.ShapeDtypeStruct((E_EDGES,), jnp.float32),
        scratch_types=[
            pltpu.VMEM((N_NODES,), jnp.float32),
            pltpu.VMEM((N_NODES,), jnp.float32),
            pltpu.VMEM((E_PER_W,), jnp.int32),
            pltpu.VMEM((E_PER_W,), jnp.int32),
            pltpu.VMEM((E_PER_W,), jnp.float32),
            pltpu.SemaphoreType.DMA,
        ],
    )


def kernel(x_mirna, x_disease, edge_label_index, W_mirna, b_mirna,
           W_disease, b_disease, W_cls, b_cls):
    wc_m = W_cls[:DIM]
    wc_d = W_cls[DIM:]
    b0 = b_cls.astype(jnp.float32).reshape(1, 1)
    s_m, s_d = _node_scores(x_mirna, x_disease, W_mirna, W_disease,
                            wc_m, wc_d, b_mirna.reshape(1, DIM),
                            b_disease.reshape(1, DIM), b0)
    ei = edge_label_index.astype(jnp.int32)
    return _edge_kernel()(s_m.reshape(-1), s_d.reshape(-1), ei)


# speedup vs baseline: 26.2213x; 1.0334x over previous
import functools

import jax
import jax.numpy as jnp
from jax import lax
from jax.experimental import pallas as pl
from jax.experimental.pallas import tpu as pltpu
from jax.experimental.pallas import tpu_sc as plsc

N_NODES = 10000
E_EDGES = 320000
DIM = 128
LANES = 16
UNROLL = 5
NUM_WORKERS = 32
E_PER_W = E_EDGES // NUM_WORKERS
ROW_BLOCK = 2000


def _scores_body(xm_ref, xd_ref, wm_ref, wd_ref, wcm_ref, wcd_ref,
                 bm_ref, bd_ref, b0_ref, om_ref, od_ref):
    f32 = jnp.float32
    i = pl.program_id(0)
    vm = jnp.dot(wm_ref[:], wcm_ref[:], preferred_element_type=f32)
    vd = jnp.dot(wd_ref[:], wcd_ref[:], preferred_element_type=f32)
    cm = jnp.dot(bm_ref[:], wcm_ref[:], preferred_element_type=f32)
    cd = jnp.dot(bd_ref[:], wcd_ref[:], preferred_element_type=f32)
    dn = (((0,), (1,)), ((), ()))
    sm = lax.dot_general(vm, xm_ref[:], dn, preferred_element_type=f32)
    sd = lax.dot_general(vd, xd_ref[:], dn, preferred_element_type=f32)
    om_ref[:] = (sm + cm + b0_ref[:]).reshape(1, 1, ROW_BLOCK)
    od_ref[:] = (sd + cd).reshape(1, 1, ROW_BLOCK)


def _node_scores(xm, xd, wm, wd, wcm, wcd, bm, bd, b0):
    n = xm.shape[0]
    fm, fd = xm.shape[1], xd.shape[1]
    return pl.pallas_call(
        _scores_body,
        grid=(n // ROW_BLOCK,),
        in_specs=[
            pl.BlockSpec((ROW_BLOCK, fm), lambda i: (i, 0)),
            pl.BlockSpec((ROW_BLOCK, fd), lambda i: (i, 0)),
            pl.BlockSpec((fm, DIM), lambda i: (0, 0)),
            pl.BlockSpec((fd, DIM), lambda i: (0, 0)),
            pl.BlockSpec((DIM, 1), lambda i: (0, 0)),
            pl.BlockSpec((DIM, 1), lambda i: (0, 0)),
            pl.BlockSpec((1, DIM), lambda i: (0, 0)),
            pl.BlockSpec((1, DIM), lambda i: (0, 0)),
            pl.BlockSpec((1, 1), lambda i: (0, 0)),
        ],
        out_specs=[
            pl.BlockSpec((1, 1, ROW_BLOCK), lambda i: (i, 0, 0)),
            pl.BlockSpec((1, 1, ROW_BLOCK), lambda i: (i, 0, 0)),
        ],
        out_shape=[
            jax.ShapeDtypeStruct((n // ROW_BLOCK, 1, ROW_BLOCK), jnp.float32),
            jax.ShapeDtypeStruct((n // ROW_BLOCK, 1, ROW_BLOCK), jnp.float32),
        ],
    )(xm, xd, wm, wd, wcm, wcd, bm, bd, b0)


def _edge_body(sm_hbm, sd_hbm, ei_hbm, out_hbm,
               sm_v, sd_v, src_v, dst_v, out_v, sem):
    wid = lax.axis_index("s") * 2 + lax.axis_index("c")
    base = wid * E_PER_W
    c1 = pltpu.async_copy(sm_hbm, sm_v, sem)
    c2 = pltpu.async_copy(sd_hbm, sd_v, sem)
    c3 = pltpu.async_copy(ei_hbm.at[0, pl.ds(base, E_PER_W)], src_v, sem)
    c4 = pltpu.async_copy(ei_hbm.at[1, pl.ds(base, E_PER_W)], dst_v, sem)
    c1.wait()
    c2.wait()
    c3.wait()
    c4.wait()

    def body(i, carry):
        chunk = i * (LANES * UNROLL)
        for u in range(UNROLL):
            off = chunk + u * LANES
            si = src_v[pl.ds(off, LANES)]
            di = dst_v[pl.ds(off, LANES)]
            a = plsc.load_gather(sm_v, [si])
            b = plsc.load_gather(sd_v, [di])
            out_v[pl.ds(off, LANES)] = a + b
        return carry

    lax.fori_loop(0, E_PER_W // (LANES * UNROLL), body, 0)
    pltpu.sync_copy(out_v, out_hbm.at[pl.ds(base, E_PER_W)])


@functools.cache
def _edge_kernel():
    return pl.kernel(
        _edge_body,
        mesh=plsc.VectorSubcoreMesh(core_axis_name="c", subcore_axis_name="s"),
        compiler_params=pltpu.CompilerParams(
            needs_layout_passes=False, use_tc_tiling_on_sc=False),
        out_type=jax.ShapeDtypeStruct((E_EDGES,), jnp.float32),
        scratch_types=[
            pltpu.VMEM((N_NODES,), jnp.float32),
            pltpu.VMEM((N_NODES,), jnp.float32),
            pltpu.VMEM((E_PER_W,), jnp.int32),
            pltpu.VMEM((E_PER_W,), jnp.int32),
            pltpu.VMEM((E_PER_W,), jnp.float32),
            pltpu.SemaphoreType.DMA,
        ],
    )


def _sigmoid_body(z_ref, o_ref):
    o_ref[:] = 1.0 / (1.0 + jnp.exp(-z_ref[:]))


def _sigmoid_tc(z):
    e = z.shape[0]
    blk = 32768
    return pl.pallas_call(
        _sigmoid_body,
        grid=((e + blk - 1) // blk,),
        in_specs=[pl.BlockSpec((blk,), lambda i: (i,))],
        out_specs=pl.BlockSpec((blk,), lambda i: (i,)),
        out_shape=jax.ShapeDtypeStruct((e,), jnp.float32),
    )(z)


def kernel(x_mirna, x_disease, edge_label_index, W_mirna, b_mirna,
           W_disease, b_disease, W_cls, b_cls):
    wc_m = W_cls[:DIM]
    wc_d = W_cls[DIM:]
    b0 = b_cls.astype(jnp.float32).reshape(1, 1)
    s_m, s_d = _node_scores(x_mirna, x_disease, W_mirna, W_disease,
                            wc_m, wc_d, b_mirna.reshape(1, DIM),
                            b_disease.reshape(1, DIM), b0)
    ei = edge_label_index.astype(jnp.int32)
    z = _edge_kernel()(s_m.reshape(-1), s_d.reshape(-1), ei)
    return _sigmoid_tc(z)


# speedup vs baseline: 26.3283x; 1.0041x over previous
import functools

import jax
import jax.numpy as jnp
from jax import lax
from jax.experimental import pallas as pl
from jax.experimental.pallas import tpu as pltpu
from jax.experimental.pallas import tpu_sc as plsc

N_NODES = 10000
E_EDGES = 320000
DIM = 128
LANES = 16
UNROLL = 1
NUM_WORKERS = 32
E_PER_W = E_EDGES // NUM_WORKERS
ROW_BLOCK = 2000


def _scores_body(xm_ref, xd_ref, wm_ref, wd_ref, wcm_ref, wcd_ref,
                 bm_ref, bd_ref, b0_ref, om_ref, od_ref):
    f32 = jnp.float32
    i = pl.program_id(0)
    vm = jnp.dot(wm_ref[:], wcm_ref[:], preferred_element_type=f32)
    vd = jnp.dot(wd_ref[:], wcd_ref[:], preferred_element_type=f32)
    cm = jnp.dot(bm_ref[:], wcm_ref[:], preferred_element_type=f32)
    cd = jnp.dot(bd_ref[:], wcd_ref[:], preferred_element_type=f32)
    dn = (((0,), (1,)), ((), ()))
    sm = lax.dot_general(vm, xm_ref[:], dn, preferred_element_type=f32)
    sd = lax.dot_general(vd, xd_ref[:], dn, preferred_element_type=f32)
    om_ref[:] = (sm + cm + b0_ref[:]).reshape(1, 1, ROW_BLOCK)
    od_ref[:] = (sd + cd).reshape(1, 1, ROW_BLOCK)


def _node_scores(xm, xd, wm, wd, wcm, wcd, bm, bd, b0):
    n = xm.shape[0]
    fm, fd = xm.shape[1], xd.shape[1]
    return pl.pallas_call(
        _scores_body,
        grid=(n // ROW_BLOCK,),
        in_specs=[
            pl.BlockSpec((ROW_BLOCK, fm), lambda i: (i, 0)),
            pl.BlockSpec((ROW_BLOCK, fd), lambda i: (i, 0)),
            pl.BlockSpec((fm, DIM), lambda i: (0, 0)),
            pl.BlockSpec((fd, DIM), lambda i: (0, 0)),
            pl.BlockSpec((DIM, 1), lambda i: (0, 0)),
            pl.BlockSpec((DIM, 1), lambda i: (0, 0)),
            pl.BlockSpec((1, DIM), lambda i: (0, 0)),
            pl.BlockSpec((1, DIM), lambda i: (0, 0)),
            pl.BlockSpec((1, 1), lambda i: (0, 0)),
        ],
        out_specs=[
            pl.BlockSpec((1, 1, ROW_BLOCK), lambda i: (i, 0, 0)),
            pl.BlockSpec((1, 1, ROW_BLOCK), lambda i: (i, 0, 0)),
        ],
        out_shape=[
            jax.ShapeDtypeStruct((n // ROW_BLOCK, 1, ROW_BLOCK), jnp.float32),
            jax.ShapeDtypeStruct((n // ROW_BLOCK, 1, ROW_BLOCK), jnp.float32),
        ],
    )(xm, xd, wm, wd, wcm, wcd, bm, bd, b0)


def _edge_body(sm_hbm, sd_hbm, ei_hbm, out_hbm,
               sm_v, sd_v, src_v, dst_v, out_v, sem):
    wid = lax.axis_index("s") * 2 + lax.axis_index("c")
    base = wid * E_PER_W
    c1 = pltpu.async_copy(sm_hbm, sm_v, sem)
    c2 = pltpu.async_copy(sd_hbm, sd_v, sem)
    c3 = pltpu.async_copy(ei_hbm.at[0, pl.ds(base, E_PER_W)], src_v, sem)
    c4 = pltpu.async_copy(ei_hbm.at[1, pl.ds(base, E_PER_W)], dst_v, sem)
    c1.wait()
    c2.wait()
    c3.wait()
    c4.wait()

    def body(i, carry):
        chunk = i * (LANES * UNROLL)
        for u in range(UNROLL):
            off = chunk + u * LANES
            si = src_v[pl.ds(off, LANES)]
            di = dst_v[pl.ds(off, LANES)]
            a = plsc.load_gather(sm_v, [si])
            b = plsc.load_gather(sd_v, [di])
            out_v[pl.ds(off, LANES)] = a + b
        return carry

    lax.fori_loop(0, E_PER_W // (LANES * UNROLL), body, 0)
    pltpu.sync_copy(out_v, out_hbm.at[pl.ds(base, E_PER_W)])


@functools.cache
def _edge_kernel():
    return pl.kernel(
        _edge_body,
        mesh=plsc.VectorSubcoreMesh(core_axis_name="c", subcore_axis_name="s"),
        compiler_params=pltpu.CompilerParams(
            needs_layout_passes=False, use_tc_tiling_on_sc=False),
        out_type=jax.ShapeDtypeStruct((E_EDGES,), jnp.float32),
        scratch_types=[
            pltpu.VMEM((N_NODES,), jnp.float32),
            pltpu.VMEM((N_NODES,), jnp.float32),
            pltpu.VMEM((E_PER_W,), jnp.int32),
            pltpu.VMEM((E_PER_W,), jnp.int32),
            pltpu.VMEM((E_PER_W,), jnp.float32),
            pltpu.SemaphoreType.DMA,
        ],
    )


def _sigmoid_body(z_ref, o_ref):
    o_ref[:] = 1.0 / (1.0 + jnp.exp(-z_ref[:]))


def _sigmoid_tc(z):
    e = z.shape[0]
    blk = 32768
    return pl.pallas_call(
        _sigmoid_body,
        grid=((e + blk - 1) // blk,),
        in_specs=[pl.BlockSpec((blk,), lambda i: (i,))],
        out_specs=pl.BlockSpec((blk,), lambda i: (i,)),
        out_shape=jax.ShapeDtypeStruct((e,), jnp.float32),
    )(z)


def kernel(x_mirna, x_disease, edge_label_index, W_mirna, b_mirna,
           W_disease, b_disease, W_cls, b_cls):
    wc_m = W_cls[:DIM]
    wc_d = W_cls[DIM:]
    b0 = b_cls.astype(jnp.float32).reshape(1, 1)
    s_m, s_d = _node_scores(x_mirna, x_disease, W_mirna, W_disease,
                            wc_m, wc_d, b_mirna.reshape(1, DIM),
                            b_disease.reshape(1, DIM), b0)
    ei = edge_label_index.astype(jnp.int32)
    z = _edge_kernel()(s_m.reshape(-1), s_d.reshape(-1), ei)
    return _sigmoid_tc(z)


# speedup vs baseline: 36.3658x; 1.3812x over previous
import functools

import jax
import jax.numpy as jnp
from jax import lax
from jax.experimental import pallas as pl
from jax.experimental.pallas import tpu as pltpu
from jax.experimental.pallas import tpu_sc as plsc

E_EDGES = 320000
DIM = 128
LANES = 16
ROW_BLOCK = 2048
N_PAD = 10240


def _scores_body(xm_ref, xd_ref, wm_ref, wd_ref, wc_ref,
                 bm_ref, bd_ref, b0_ref, om_ref, od_ref):
    f32 = jnp.float32
    wcm = wc_ref[pl.ds(0, DIM), :]
    wcd = wc_ref[pl.ds(DIM, DIM), :]
    vm = jnp.dot(wm_ref[:], wcm, preferred_element_type=f32)
    vd = jnp.dot(wd_ref[:], wcd, preferred_element_type=f32)
    cm = jnp.dot(bm_ref[:], wcm, preferred_element_type=f32)
    cd = jnp.dot(bd_ref[:], wcd, preferred_element_type=f32)
    dn = (((0,), (1,)), ((), ()))
    sm = lax.dot_general(vm, xm_ref[:], dn, preferred_element_type=f32)
    sd = lax.dot_general(vd, xd_ref[:], dn, preferred_element_type=f32)
    om_ref[:] = (sm + cm + b0_ref[:]).reshape(1, 1, ROW_BLOCK)
    od_ref[:] = (sd + cd).reshape(1, 1, ROW_BLOCK)


def _node_scores(xm, xd, wm, wd, wc, bm, bd, b0):
    fm, fd = xm.shape[1], xd.shape[1]
    nb = N_PAD // ROW_BLOCK
    return pl.pallas_call(
        _scores_body,
        grid=(nb,),
        in_specs=[
            pl.BlockSpec((ROW_BLOCK, fm), lambda i: (i, 0)),
            pl.BlockSpec((ROW_BLOCK, fd), lambda i: (i, 0)),
            pl.BlockSpec((fm, DIM), lambda i: (0, 0)),
            pl.BlockSpec((fd, DIM), lambda i: (0, 0)),
            pl.BlockSpec((2 * DIM, 1), lambda i: (0, 0)),
            pl.BlockSpec((1, DIM), lambda i: (0, 0)),
            pl.BlockSpec((1, DIM), lambda i: (0, 0)),
            pl.BlockSpec((1, 1), lambda i: (0, 0)),
        ],
        out_specs=[
            pl.BlockSpec((1, 1, ROW_BLOCK), lambda i: (i, 0, 0)),
            pl.BlockSpec((1, 1, ROW_BLOCK), lambda i: (i, 0, 0)),
        ],
        out_shape=[
            jax.ShapeDtypeStruct((nb, 1, ROW_BLOCK), jnp.float32),
            jax.ShapeDtypeStruct((nb, 1, ROW_BLOCK), jnp.float32),
        ],
    )(xm, xd, wm, wd, wc, bm, bd, b0)


N_TILES = E_EDGES // 128
TILES_PER_W = 79
CHUNKS_PER_W = TILES_PER_W * 8


def _edge_body(sm_hbm, sd_hbm, ei_hbm, out_hbm,
               sm_v, sd_v, src_v, dst_v, out_v, sem):
    wid = lax.axis_index("s") * 2 + lax.axis_index("c")
    rem = N_TILES - 32 * (N_TILES // 32)
    start = jnp.minimum(wid * (N_TILES // 32) + jnp.minimum(wid, rem),
                        N_TILES - TILES_PER_W)
    c1 = pltpu.async_copy(sm_hbm, sm_v, sem)
    c2 = pltpu.async_copy(sd_hbm, sd_v, sem)
    c3 = pltpu.async_copy(ei_hbm.at[pl.ds(start, TILES_PER_W), pl.ds(0, 1), :],
                          src_v, sem)
    c4 = pltpu.async_copy(ei_hbm.at[pl.ds(start, TILES_PER_W), pl.ds(1, 1), :],
                          dst_v, sem)
    c1.wait()
    c2.wait()
    c3.wait()
    c4.wait()

    @plsc.parallel_loop(0, CHUNKS_PER_W, unroll=4)
    def body(i):
        t = i >> 3
        off = (i & 7) * LANES
        si = src_v[t, 0, pl.ds(off, LANES)]
        di = dst_v[t, 0, pl.ds(off, LANES)]
        a = plsc.load_gather(sm_v, [si])
        b = plsc.load_gather(sd_v, [di])
        z = a + b
        out_v[pl.ds(i * LANES, LANES)] = 1.0 / (1.0 + jnp.exp(-z))
    pltpu.sync_copy(out_v, out_hbm.at[pl.ds(start * 128, TILES_PER_W * 128)])


@functools.cache
def _edge_kernel():
    return pl.kernel(
        _edge_body,
        mesh=plsc.VectorSubcoreMesh(core_axis_name="c", subcore_axis_name="s"),
        compiler_params=pltpu.CompilerParams(
            needs_layout_passes=False, use_tc_tiling_on_sc=False),
        out_type=jax.ShapeDtypeStruct((E_EDGES,), jnp.float32),
        scratch_types=[
            pltpu.VMEM((N_PAD,), jnp.float32),
            pltpu.VMEM((N_PAD,), jnp.float32),
            pltpu.VMEM((TILES_PER_W, 1, 128), jnp.int32),
            pltpu.VMEM((TILES_PER_W, 1, 128), jnp.int32),
            pltpu.VMEM((TILES_PER_W * 128,), jnp.float32),
            pltpu.SemaphoreType.DMA,
        ],
    )


def kernel(x_mirna, x_disease, edge_label_index, W_mirna, b_mirna,
           W_disease, b_disease, W_cls, b_cls):
    b0 = b_cls.astype(jnp.float32).reshape(1, 1)
    s_m, s_d = _node_scores(x_mirna, x_disease, W_mirna, W_disease,
                            W_cls, b_mirna.reshape(1, DIM),
                            b_disease.reshape(1, DIM), b0)
    ei = edge_label_index.astype(jnp.int32)
    ei3 = ei.reshape(2, N_TILES, 128).transpose(1, 0, 2)
    return _edge_kernel()(s_m.reshape(-1), s_d.reshape(-1), ei3)
